# baseline (speedup 1.0000x reference)
import functools

import jax
import jax.numpy as jnp
from jax import lax
from jax.experimental import pallas as pl
from jax.experimental.pallas import tpu as pltpu
from jax.experimental.pallas import tpu_sc as plsc

F = 100000
C = 128
FP = 102400
NW = 32
BPW = FP // NW
CH = 128
NCH = BPW // CH
BF = 1024


def _face_adjacency(faces):
    e = jnp.concatenate([faces[:, [0, 1]], faces[:, [1, 2]], faces[:, [2, 0]]],
                        axis=0)
    e = jnp.sort(e, axis=1)
    key = (e[:, 0].astype(jnp.uint32) << 16) | e[:, 1].astype(jnp.uint32)
    idx = jnp.arange(3 * F, dtype=jnp.int32)
    sorted_key, order = lax.sort((key, idx), num_keys=1, is_stable=True)
    match = sorted_key[:-1] == sorted_key[1:]
    a = order[:-1]
    b = order[1:]
    fa = a % F
    sa = a // F
    fb = b % F
    sb = b // F
    CKP = jnp.tile(jnp.arange(F, dtype=jnp.int32)[:, None], (1, 3))
    row_a = jnp.where(match, fa, F)
    CKP = CKP.at[row_a, sa].set(fb, mode='drop')
    row_b = jnp.where(match, fb, F)
    CKP = CKP.at[row_b, sb].set(fa, mode='drop')
    return CKP


def _sc_gather3(ff_pad, idx3):
    mesh = plsc.VectorSubcoreMesh(core_axis_name="c", subcore_axis_name="s")

    @functools.partial(
        pl.kernel, mesh=mesh,
        out_type=jax.ShapeDtypeStruct((3, FP, C), jnp.float32),
        scratch_types=[
            pltpu.VMEM((CH,), jnp.int32),
            pltpu.VMEM((CH, C), jnp.float32),
            pltpu.SemaphoreType.DMA,
        ],
    )
    def gather_kernel(ff_hbm, idx_hbm, out_hbm, idx_v, rows_v, sem):
        wid = lax.axis_index("s") * 2 + lax.axis_index("c")
        base = wid * BPW
        for k in range(3):
            def body(c, _):
                off = base + c * CH
                pltpu.sync_copy(idx_hbm.at[pl.ds(k * FP + off, CH)], idx_v)
                pltpu.async_copy(ff_hbm.at[idx_v], rows_v, sem).wait()
                pltpu.sync_copy(rows_v, out_hbm.at[k, pl.ds(off, CH)])
                return _
            lax.fori_loop(0, NCH, body, 0)

    return gather_kernel(ff_pad, idx3)


def _tc_mix(ff_pad, nbr, wt, b2):
    def body(x_ref, n_ref, w_ref, b_ref, o_ref):
        y = x_ref[...]
        n0 = n_ref[0]
        n1 = n_ref[1]
        n2 = n_ref[2]
        f1 = n0 + n1 + n2
        f2 = jnp.abs(n2 - n0) + jnp.abs(n0 - n1) + jnp.abs(n1 - n2)
        f3 = jnp.abs(y - n0) + jnp.abs(y - n1) + jnp.abs(y - n2)
        acc = jnp.dot(y, w_ref[0:C, :], preferred_element_type=jnp.float32)
        acc += jnp.dot(f1, w_ref[C:2 * C, :], preferred_element_type=jnp.float32)
        acc += jnp.dot(f2, w_ref[2 * C:3 * C, :], preferred_element_type=jnp.float32)
        acc += jnp.dot(f3, w_ref[3 * C:4 * C, :], preferred_element_type=jnp.float32)
        o_ref[...] = acc + b_ref[...]

    return pl.pallas_call(
        body,
        grid=(FP // BF,),
        in_specs=[
            pl.BlockSpec((BF, C), lambda i: (i, 0)),
            pl.BlockSpec((3, BF, C), lambda i: (0, i, 0)),
            pl.BlockSpec((4 * C, C), lambda i: (0, 0)),
            pl.BlockSpec((1, C), lambda i: (0, 0)),
        ],
        out_specs=pl.BlockSpec((BF, C), lambda i: (i, 0)),
        out_shape=jax.ShapeDtypeStruct((FP, C), jnp.float32),
    )(ff_pad, nbr, wt, b2)


def _adjacency_sort_only(faces):
    e = jnp.concatenate([faces[:, [0, 1]], faces[:, [1, 2]], faces[:, [2, 0]]],
                        axis=0)
    e = jnp.sort(e, axis=1)
    key = (e[:, 0].astype(jnp.uint32) << 16) | e[:, 1].astype(jnp.uint32)
    idx = jnp.arange(3 * F, dtype=jnp.int32)
    sorted_key, order = lax.sort((key, idx), num_keys=1, is_stable=True)
    match = sorted_key[:-1] == sorted_key[1:]
    a = order[:-1]
    b = order[1:]
    fa = a % F
    row_a = jnp.where(match, fa, F)
    keep = jnp.minimum(jnp.minimum(row_a.min(), b.min()), 0).astype(jnp.int32)
    return jnp.tile(jnp.arange(F, dtype=jnp.int32)[:, None], (1, 3)) + keep


def kernel(faces, face_features, W, b):
    CKP = _adjacency_sort_only(faces)
    ff_pad = jnp.zeros((FP, C), jnp.float32).at[:F].set(face_features)
    idx3 = jnp.zeros((3, FP), jnp.int32).at[:, :F].set(CKP.T).reshape(3 * FP)
    nbr = _sc_gather3(ff_pad, idx3)
    wt = jnp.transpose(W[:, :, 0, :], (2, 1, 0)).reshape(4 * C, C)
    out = _tc_mix(ff_pad, nbr, wt, b[None, :])
    return out[:F]


# speedup vs baseline: 1.3949x; 1.3949x over previous
import functools

import jax
import jax.numpy as jnp
from jax import lax
from jax.experimental import pallas as pl
from jax.experimental.pallas import tpu as pltpu
from jax.experimental.pallas import tpu_sc as plsc

F = 100000
C = 128
E = 3 * F
FP = 102400
NW = 32
CH = 128
EP = 303104
BPW = EP // NW
NCH = BPW // CH
BF = 1024


def _edge_val_dest(faces):
    e = jnp.concatenate([faces[:, [0, 1]], faces[:, [1, 2]], faces[:, [2, 0]]],
                        axis=0)
    e = jnp.sort(e, axis=1)
    key = (e[:, 0].astype(jnp.uint32) << 16) | e[:, 1].astype(jnp.uint32)
    idx = jnp.arange(E, dtype=jnp.int32)
    sorted_key, order = lax.sort((key, idx), num_keys=1, is_stable=True)
    match = sorted_key[:-1] == sorted_key[1:]
    has_pred = jnp.concatenate([jnp.zeros((1,), jnp.bool_), match])
    has_succ = jnp.concatenate([match, jnp.zeros((1,), jnp.bool_)])
    face = order % F
    fprev = jnp.concatenate([face[:1], face[:-1]])
    fnext = jnp.concatenate([face[1:], face[-1:]])
    val = jnp.where(has_pred, fprev, jnp.where(has_succ, fnext, face))
    dest = (order // F) * FP + face
    npad = EP - E
    j = jnp.arange(npad, dtype=jnp.int32)
    val_pad = (j * 37) % F
    dest_pad = 2 * FP + F + (j % (FP - F))
    return (jnp.concatenate([val, val_pad]),
            jnp.concatenate([dest, dest_pad]))


def _sc_gather_scatter(ff_pad, val, dest):
    mesh = plsc.VectorSubcoreMesh(core_axis_name="c", subcore_axis_name="s")

    @functools.partial(
        pl.kernel, mesh=mesh,
        out_type=jax.ShapeDtypeStruct((3 * FP, C), jnp.float32),
        scratch_types=[
            pltpu.VMEM((CH,), jnp.int32),
            pltpu.VMEM((CH,), jnp.int32),
            pltpu.VMEM((CH, C), jnp.float32),
            pltpu.SemaphoreType.DMA,
        ],
    )
    def gs_kernel(ff_hbm, val_hbm, dest_hbm, out_hbm, val_v, dest_v, rows_v,
                  sem):
        wid = lax.axis_index("s") * 2 + lax.axis_index("c")
        base = wid * BPW

        def body(c, carry):
            off = base + c * CH
            pltpu.sync_copy(val_hbm.at[pl.ds(off, CH)], val_v)
            pltpu.sync_copy(dest_hbm.at[pl.ds(off, CH)], dest_v)
            pltpu.async_copy(ff_hbm.at[val_v], rows_v, sem).wait()
            pltpu.async_copy(rows_v, out_hbm.at[dest_v], sem).wait()
            return carry

        lax.fori_loop(0, NCH, body, 0)

    return gs_kernel(ff_pad, val, dest)


def _tc_mix(ff_pad, nbr, wt, b2):
    def body(x_ref, n_ref, w_ref, b_ref, o_ref):
        y = x_ref[...]
        n0 = n_ref[0]
        n1 = n_ref[1]
        n2 = n_ref[2]
        f1 = n0 + n1 + n2
        f2 = jnp.abs(n2 - n0) + jnp.abs(n0 - n1) + jnp.abs(n1 - n2)
        f3 = jnp.abs(y - n0) + jnp.abs(y - n1) + jnp.abs(y - n2)
        acc = jnp.dot(y, w_ref[0:C, :], preferred_element_type=jnp.float32)
        acc += jnp.dot(f1, w_ref[C:2 * C, :], preferred_element_type=jnp.float32)
        acc += jnp.dot(f2, w_ref[2 * C:3 * C, :], preferred_element_type=jnp.float32)
        acc += jnp.dot(f3, w_ref[3 * C:4 * C, :], preferred_element_type=jnp.float32)
        o_ref[...] = acc + b_ref[...]

    return pl.pallas_call(
        body,
        grid=(FP // BF,),
        in_specs=[
            pl.BlockSpec((BF, C), lambda i: (i, 0)),
            pl.BlockSpec((3, BF, C), lambda i: (0, i, 0)),
            pl.BlockSpec((4 * C, C), lambda i: (0, 0)),
            pl.BlockSpec((1, C), lambda i: (0, 0)),
        ],
        out_specs=pl.BlockSpec((BF, C), lambda i: (i, 0)),
        out_shape=jax.ShapeDtypeStruct((FP, C), jnp.float32),
    )(ff_pad, nbr, wt, b2)


def kernel(faces, face_features, W, b):
    val, dest = _edge_val_dest(faces)
    ff_pad = jnp.zeros((FP, C), jnp.float32).at[:F].set(face_features)
    nbr = _sc_gather_scatter(ff_pad, val, dest).reshape(3, FP, C)
    wt = jnp.transpose(W[:, :, 0, :], (2, 1, 0)).reshape(4 * C, C)
    out = _tc_mix(ff_pad, nbr, wt, b[None, :])
    return out[:F]


# speedup vs baseline: 1.6209x; 1.1620x over previous
import functools

import jax
import jax.numpy as jnp
from jax import lax
from jax.experimental import pallas as pl
from jax.experimental.pallas import tpu as pltpu
from jax.experimental.pallas import tpu_sc as plsc

F = 100000
C = 128
E = 3 * F
FP = 102400
NW = 32
CH = 128
EP = 303104
BPW = EP // NW
NCH = BPW // CH
BF = 1000


def _edge_val_dest(faces):
    e = jnp.concatenate([faces[:, [0, 1]], faces[:, [1, 2]], faces[:, [2, 0]]],
                        axis=0)
    lo = jnp.minimum(e[:, 0], e[:, 1]).astype(jnp.uint32)
    hi = jnp.maximum(e[:, 0], e[:, 1]).astype(jnp.uint32)
    key = (lo << 16) | hi
    idx = jnp.arange(E, dtype=jnp.int32)
    sorted_key, order = lax.sort((key, idx), num_keys=1, is_stable=True)
    match = sorted_key[:-1] == sorted_key[1:]
    has_pred = jnp.concatenate([jnp.zeros((1,), jnp.bool_), match])
    has_succ = jnp.concatenate([match, jnp.zeros((1,), jnp.bool_)])
    face = order % F
    fprev = jnp.concatenate([face[:1], face[:-1]])
    fnext = jnp.concatenate([face[1:], face[-1:]])
    val = jnp.where(has_pred, fprev, jnp.where(has_succ, fnext, face))
    dest = (order // F) * FP + face
    npad = EP - E
    j = jnp.arange(npad, dtype=jnp.int32)
    val_pad = (j * 37) % F
    dest_pad = 2 * FP + F + (j % (FP - F))
    return (jnp.concatenate([val, val_pad]),
            jnp.concatenate([dest, dest_pad]))


def _sc_gather_scatter(ff, val, dest):
    mesh = plsc.VectorSubcoreMesh(core_axis_name="c", subcore_axis_name="s")

    @functools.partial(
        pl.kernel, mesh=mesh,
        out_type=jax.ShapeDtypeStruct((3 * FP, C), jnp.float32),
        scratch_types=[
            pltpu.VMEM((2, CH), jnp.int32),
            pltpu.VMEM((2, CH), jnp.int32),
            pltpu.VMEM((2, CH, C), jnp.float32),
            pltpu.SemaphoreType.DMA,
            pltpu.SemaphoreType.DMA,
            pltpu.SemaphoreType.DMA,
            pltpu.SemaphoreType.DMA,
        ],
    )
    def gs_kernel(ff_hbm, val_hbm, dest_hbm, out_hbm, val_v, dest_v, rows_v,
                  sem_g0, sem_g1, sem_s0, sem_s1):
        wid = lax.axis_index("s") * 2 + lax.axis_index("c")
        base = wid * BPW
        sem_g = (sem_g0, sem_g1)
        sem_s = (sem_s0, sem_s1)

        def gath(c, b):
            return pltpu.make_async_copy(
                ff_hbm.at[val_v.at[b]], rows_v.at[b], sem_g[b])

        def scat(b):
            return pltpu.make_async_copy(
                rows_v.at[b], out_hbm.at[dest_v.at[b]], sem_s[b])

        def load_idx(c, b):
            off = base + c * CH
            pltpu.sync_copy(val_hbm.at[pl.ds(off, CH)], val_v.at[b])
            pltpu.sync_copy(dest_hbm.at[pl.ds(off, CH)], dest_v.at[b])

        load_idx(0, 0)
        gath(0, 0).start()

        def body(t2, carry):
            for b in (0, 1):
                c = 2 * t2 + b
                b1 = 1 - b
                gath(c, b).wait()

                @pl.when(c >= 1)
                def _():
                    scat(b1).wait()

                scat(b).start()

                @pl.when(c + 1 < NCH)
                def _():
                    load_idx(c + 1, b1)
                    gath(c + 1, b1).start()
            return carry

        lax.fori_loop(0, NCH // 2, body, 0)
        scat(1).wait()

    return gs_kernel(ff, val, dest)


def _tc_mix(ff, nbr, wt, b2):
    def body(x_ref, n_ref, w_ref, b_ref, o_ref):
        y = x_ref[...]
        n0 = n_ref[0]
        n1 = n_ref[1]
        n2 = n_ref[2]
        f1 = n0 + n1 + n2
        f2 = jnp.abs(n2 - n0) + jnp.abs(n0 - n1) + jnp.abs(n1 - n2)
        f3 = jnp.abs(y - n0) + jnp.abs(y - n1) + jnp.abs(y - n2)
        acc = jnp.dot(y, w_ref[0:C, :], preferred_element_type=jnp.float32)
        acc += jnp.dot(f1, w_ref[C:2 * C, :], preferred_element_type=jnp.float32)
        acc += jnp.dot(f2, w_ref[2 * C:3 * C, :], preferred_element_type=jnp.float32)
        acc += jnp.dot(f3, w_ref[3 * C:4 * C, :], preferred_element_type=jnp.float32)
        o_ref[...] = acc + b_ref[...]

    return pl.pallas_call(
        body,
        grid=(F // BF,),
        in_specs=[
            pl.BlockSpec((BF, C), lambda i: (i, 0)),
            pl.BlockSpec((3, BF, C), lambda i: (0, i, 0)),
            pl.BlockSpec((4 * C, C), lambda i: (0, 0)),
            pl.BlockSpec((1, C), lambda i: (0, 0)),
        ],
        out_specs=pl.BlockSpec((BF, C), lambda i: (i, 0)),
        out_shape=jax.ShapeDtypeStruct((F, C), jnp.float32),
    )(ff, nbr, wt, b2)


def kernel(faces, face_features, W, b):
    val, dest = _edge_val_dest(faces)
    nbr = _sc_gather_scatter(face_features, val, dest).reshape(3, FP, C)
    wt = jnp.transpose(W[:, :, 0, :], (2, 1, 0)).reshape(4 * C, C)
    out = _tc_mix(face_features, nbr, wt, b[None, :])
    return out


# speedup vs baseline: 1.8456x; 1.1387x over previous
import functools

import jax
import jax.numpy as jnp
from jax import lax
from jax.experimental import pallas as pl
from jax.experimental.pallas import tpu as pltpu
from jax.experimental.pallas import tpu_sc as plsc

F = 100000
C = 128
E = 3 * F
FP = 102400
NW = 32
CH = 128
EP = 311296
BPW = EP // NW
NCH = BPW // CH
NBUF = 4
BF = 1000


def _edge_val_dest(faces):
    e = jnp.concatenate([faces[:, [0, 1]], faces[:, [1, 2]], faces[:, [2, 0]]],
                        axis=0)
    lo = jnp.minimum(e[:, 0], e[:, 1]).astype(jnp.uint32)
    hi = jnp.maximum(e[:, 0], e[:, 1]).astype(jnp.uint32)
    key = (lo << 16) | hi
    idx = jnp.arange(E, dtype=jnp.int32)
    sorted_key, order = lax.sort((key, idx), num_keys=1, is_stable=True)
    match = sorted_key[:-1] == sorted_key[1:]
    has_pred = jnp.concatenate([jnp.zeros((1,), jnp.bool_), match])
    has_succ = jnp.concatenate([match, jnp.zeros((1,), jnp.bool_)])
    face = order % F
    fprev = jnp.concatenate([face[:1], face[:-1]])
    fnext = jnp.concatenate([face[1:], face[-1:]])
    val = jnp.where(has_pred, fprev, jnp.where(has_succ, fnext, face))
    dest = (order // F) * FP + face
    npad = EP - E
    j = jnp.arange(npad, dtype=jnp.int32)
    val_pad = (j * 37) % F
    dest_pad = 2 * FP + F + (j % (FP - F))
    return (jnp.concatenate([val, val_pad]),
            jnp.concatenate([dest, dest_pad]))


def _sc_gather_scatter(ff, val, dest):
    mesh = plsc.VectorSubcoreMesh(core_axis_name="c", subcore_axis_name="s")

    @functools.partial(
        pl.kernel, mesh=mesh,
        out_type=jax.ShapeDtypeStruct((3 * FP, C), jnp.float32),
        scratch_types=[
            pltpu.VMEM((NCH, CH), jnp.int32),
            pltpu.VMEM((NCH, CH), jnp.int32),
            pltpu.VMEM((NBUF, CH, C), jnp.float32),
            pltpu.SemaphoreType.DMA,
            pltpu.SemaphoreType.DMA,
            pltpu.SemaphoreType.DMA,
            pltpu.SemaphoreType.DMA,
            pltpu.SemaphoreType.DMA,
            pltpu.SemaphoreType.DMA,
            pltpu.SemaphoreType.DMA,
            pltpu.SemaphoreType.DMA,
        ],
    )
    def gs_kernel(ff_hbm, val_hbm, dest_hbm, out_hbm, val_v, dest_v, rows_v,
                  g0, g1, g2, g3, s0, s1, s2, s3):
        wid = lax.axis_index("s") * 2 + lax.axis_index("c")
        sem_g = (g0, g1, g2, g3)
        sem_s = (s0, s1, s2, s3)

        pltpu.sync_copy(val_hbm.at[wid], val_v)
        pltpu.sync_copy(dest_hbm.at[wid], dest_v)

        def gath(c, b):
            return pltpu.make_async_copy(
                ff_hbm.at[val_v.at[c]], rows_v.at[b], sem_g[b])

        def scat(c, b):
            return pltpu.make_async_copy(
                rows_v.at[b], out_hbm.at[dest_v.at[c]], sem_s[b])

        for c0 in range(NBUF - 1):
            gath(c0, c0).start()

        def body(t4, carry):
            for b in range(NBUF):
                c = NBUF * t4 + b
                bp = (b + NBUF - 1) % NBUF
                gath(c, b).wait()

                @pl.when(c >= 1)
                def _():
                    scat(c - 1, bp).wait()

                scat(c, b).start()

                @pl.when(c + NBUF - 1 < NCH)
                def _():
                    gath(c + NBUF - 1, bp).start()
            return carry

        lax.fori_loop(0, NCH // NBUF, body, 0)
        scat(NCH - 1, (NCH - 1) % NBUF).wait()

    return gs_kernel(ff, val, dest)


def _tc_mix(ff, nbr, wt, b2):
    def body(x_ref, n_ref, w_ref, b_ref, o_ref):
        y = x_ref[...]
        n0 = n_ref[0]
        n1 = n_ref[1]
        n2 = n_ref[2]
        f1 = n0 + n1 + n2
        f2 = jnp.abs(n2 - n0) + jnp.abs(n0 - n1) + jnp.abs(n1 - n2)
        f3 = jnp.abs(y - n0) + jnp.abs(y - n1) + jnp.abs(y - n2)
        acc = jnp.dot(y, w_ref[0:C, :], preferred_element_type=jnp.float32)
        acc += jnp.dot(f1, w_ref[C:2 * C, :], preferred_element_type=jnp.float32)
        acc += jnp.dot(f2, w_ref[2 * C:3 * C, :], preferred_element_type=jnp.float32)
        acc += jnp.dot(f3, w_ref[3 * C:4 * C, :], preferred_element_type=jnp.float32)
        o_ref[...] = acc + b_ref[...]

    return pl.pallas_call(
        body,
        grid=(F // BF,),
        in_specs=[
            pl.BlockSpec((BF, C), lambda i: (i, 0)),
            pl.BlockSpec((3, BF, C), lambda i: (0, i, 0)),
            pl.BlockSpec((4 * C, C), lambda i: (0, 0)),
            pl.BlockSpec((1, C), lambda i: (0, 0)),
        ],
        out_specs=pl.BlockSpec((BF, C), lambda i: (i, 0)),
        out_shape=jax.ShapeDtypeStruct((F, C), jnp.float32),
    )(ff, nbr, wt, b2)


def kernel(faces, face_features, W, b):
    val, dest = _edge_val_dest(faces)
    val = val.reshape(NW, NCH, CH)
    dest = dest.reshape(NW, NCH, CH)
    nbr = _sc_gather_scatter(face_features, val, dest).reshape(3, FP, C)
    wt = jnp.transpose(W[:, :, 0, :], (2, 1, 0)).reshape(4 * C, C)
    out = _tc_mix(face_features, nbr, wt, b[None, :])
    return out


# speedup vs baseline: 1.9204x; 1.0405x over previous
import functools

import jax
import jax.numpy as jnp
from jax import lax
from jax.experimental import pallas as pl
from jax.experimental.pallas import tpu as pltpu
from jax.experimental.pallas import tpu_sc as plsc

F = 100000
C = 128
E = 3 * F
FP = 102400
NW = 32
CH = 128
EP = 311296
BPW = EP // NW
NCH = BPW // CH
NBUF = 4
BF = 1000


def _edge_val_dest(faces):
    e = jnp.concatenate([faces[:, [0, 1]], faces[:, [1, 2]], faces[:, [2, 0]]],
                        axis=0)
    lo = jnp.minimum(e[:, 0], e[:, 1]).astype(jnp.uint32)
    hi = jnp.maximum(e[:, 0], e[:, 1]).astype(jnp.uint32)
    key = (lo << 16) | hi
    idx = jnp.arange(E, dtype=jnp.int32)
    dest_e = (idx // F) * FP + (idx % F)
    sorted_key, dest = lax.sort((key, dest_e), num_keys=2, is_stable=False)
    match = sorted_key[:-1] == sorted_key[1:]
    has_pred = jnp.concatenate([jnp.zeros((1,), jnp.bool_), match])
    has_succ = jnp.concatenate([match, jnp.zeros((1,), jnp.bool_)])
    face = dest % FP
    fprev = jnp.concatenate([face[:1], face[:-1]])
    fnext = jnp.concatenate([face[1:], face[-1:]])
    val = jnp.where(has_pred, fprev, jnp.where(has_succ, fnext, face))
    npad = EP - E
    j = jnp.arange(npad, dtype=jnp.int32)
    val_pad = (j * 37) % F
    dest_pad = 2 * FP + F + (j % (FP - F))
    return (jnp.concatenate([val, val_pad]),
            jnp.concatenate([dest, dest_pad]))


def _sc_gather_scatter(ff, val, dest):
    mesh = plsc.VectorSubcoreMesh(core_axis_name="c", subcore_axis_name="s")

    @functools.partial(
        pl.kernel, mesh=mesh,
        out_type=jax.ShapeDtypeStruct((3 * FP, C), jnp.float32),
        scratch_types=[
            pltpu.VMEM((NCH, CH), jnp.int32),
            pltpu.VMEM((NCH, CH), jnp.int32),
            pltpu.VMEM((NBUF, CH, C), jnp.float32),
            pltpu.SemaphoreType.DMA,
            pltpu.SemaphoreType.DMA,
            pltpu.SemaphoreType.DMA,
            pltpu.SemaphoreType.DMA,
            pltpu.SemaphoreType.DMA,
            pltpu.SemaphoreType.DMA,
            pltpu.SemaphoreType.DMA,
            pltpu.SemaphoreType.DMA,
        ],
    )
    def gs_kernel(ff_hbm, val_hbm, dest_hbm, out_hbm, val_v, dest_v, rows_v,
                  g0, g1, g2, g3, s0, s1, s2, s3):
        wid = lax.axis_index("s") * 2 + lax.axis_index("c")
        sem_g = (g0, g1, g2, g3)
        sem_s = (s0, s1, s2, s3)

        pltpu.sync_copy(val_hbm.at[wid], val_v)
        pltpu.sync_copy(dest_hbm.at[wid], dest_v)

        def gath(c, b):
            return pltpu.make_async_copy(
                ff_hbm.at[val_v.at[c]], rows_v.at[b], sem_g[b])

        def scat(c, b):
            return pltpu.make_async_copy(
                rows_v.at[b], out_hbm.at[dest_v.at[c]], sem_s[b])

        for c0 in range(NBUF - 1):
            gath(c0, c0).start()

        def body(t4, carry):
            for b in range(NBUF):
                c = NBUF * t4 + b
                bp = (b + NBUF - 1) % NBUF
                gath(c, b).wait()

                @pl.when(c >= 1)
                def _():
                    scat(c - 1, bp).wait()

                scat(c, b).start()

                @pl.when(c + NBUF - 1 < NCH)
                def _():
                    gath(c + NBUF - 1, bp).start()
            return carry

        lax.fori_loop(0, NCH // NBUF, body, 0)
        scat(NCH - 1, (NCH - 1) % NBUF).wait()

    return gs_kernel(ff, val, dest)


def _tc_mix(ff, nbr, wt, b2):
    def body(x_ref, n_ref, w_ref, b_ref, o_ref):
        y = x_ref[...]
        n0 = n_ref[0]
        n1 = n_ref[1]
        n2 = n_ref[2]
        f1 = n0 + n1 + n2
        f2 = jnp.abs(n2 - n0) + jnp.abs(n0 - n1) + jnp.abs(n1 - n2)
        f3 = jnp.abs(y - n0) + jnp.abs(y - n1) + jnp.abs(y - n2)
        acc = jnp.dot(y, w_ref[0:C, :], preferred_element_type=jnp.float32)
        acc += jnp.dot(f1, w_ref[C:2 * C, :], preferred_element_type=jnp.float32)
        acc += jnp.dot(f2, w_ref[2 * C:3 * C, :], preferred_element_type=jnp.float32)
        acc += jnp.dot(f3, w_ref[3 * C:4 * C, :], preferred_element_type=jnp.float32)
        o_ref[...] = acc + b_ref[...]

    return pl.pallas_call(
        body,
        grid=(F // BF,),
        in_specs=[
            pl.BlockSpec((BF, C), lambda i: (i, 0)),
            pl.BlockSpec((3, BF, C), lambda i: (0, i, 0)),
            pl.BlockSpec((4 * C, C), lambda i: (0, 0)),
            pl.BlockSpec((1, C), lambda i: (0, 0)),
        ],
        out_specs=pl.BlockSpec((BF, C), lambda i: (i, 0)),
        out_shape=jax.ShapeDtypeStruct((F, C), jnp.float32),
    )(ff, nbr, wt, b2)


def kernel(faces, face_features, W, b):
    val, dest = _edge_val_dest(faces)
    val = val.reshape(NW, NCH, CH)
    dest = dest.reshape(NW, NCH, CH)
    nbr = _sc_gather_scatter(face_features, val, dest).reshape(3, FP, C)
    wt = jnp.transpose(W[:, :, 0, :], (2, 1, 0)).reshape(4 * C, C)
    out = _tc_mix(face_features, nbr, wt, b[None, :])
    return out


# speedup vs baseline: 2.0616x; 1.0736x over previous
import functools

import jax
import jax.numpy as jnp
from jax import lax
from jax.experimental import pallas as pl
from jax.experimental.pallas import tpu as pltpu
from jax.experimental.pallas import tpu_sc as plsc

F = 100000
C = 128
E = 3 * F
FP = 102400
NW = 32
CH = 128
EP = 311296
BPW = EP // NW
NCH = BPW // CH
NBUF = 4
BF = 2000


def _edge_val_dest(faces):
    e = jnp.concatenate([faces[:, [0, 1]], faces[:, [1, 2]], faces[:, [2, 0]]],
                        axis=0)
    lo = jnp.minimum(e[:, 0], e[:, 1]).astype(jnp.uint32)
    hi = jnp.maximum(e[:, 0], e[:, 1]).astype(jnp.uint32)
    key = (lo << 16) | hi
    idx = jnp.arange(E, dtype=jnp.int32)
    dest_e = (idx // F) * FP + (idx % F)
    sorted_key, dest = lax.sort((key, dest_e), num_keys=2, is_stable=False)
    match = sorted_key[:-1] == sorted_key[1:]
    has_pred = jnp.concatenate([jnp.zeros((1,), jnp.bool_), match])
    has_succ = jnp.concatenate([match, jnp.zeros((1,), jnp.bool_)])
    face = dest % FP
    fprev = jnp.concatenate([face[:1], face[:-1]])
    fnext = jnp.concatenate([face[1:], face[-1:]])
    val = jnp.where(has_pred, fprev, jnp.where(has_succ, fnext, face))
    npad = EP - E
    j = jnp.arange(npad, dtype=jnp.int32)
    val_pad = (j * 37) % F
    dest_pad = 2 * FP + F + (j % (FP - F))
    return (jnp.concatenate([val, val_pad]),
            jnp.concatenate([dest, dest_pad]))


def _sc_gather_scatter(ff, val, dest):
    mesh = plsc.VectorSubcoreMesh(core_axis_name="c", subcore_axis_name="s")

    @functools.partial(
        pl.kernel, mesh=mesh,
        out_type=jax.ShapeDtypeStruct((3 * FP, C), jnp.float32),
        scratch_types=[
            pltpu.VMEM((NCH, CH), jnp.int32),
            pltpu.VMEM((NCH, CH), jnp.int32),
            pltpu.VMEM((NBUF, CH, C), jnp.float32),
            pltpu.SemaphoreType.DMA,
            pltpu.SemaphoreType.DMA,
            pltpu.SemaphoreType.DMA,
            pltpu.SemaphoreType.DMA,
            pltpu.SemaphoreType.DMA,
            pltpu.SemaphoreType.DMA,
            pltpu.SemaphoreType.DMA,
            pltpu.SemaphoreType.DMA,
        ],
    )
    def gs_kernel(ff_hbm, val_hbm, dest_hbm, out_hbm, val_v, dest_v, rows_v,
                  g0, g1, g2, g3, s0, s1, s2, s3):
        wid = lax.axis_index("s") * 2 + lax.axis_index("c")
        sem_g = (g0, g1, g2, g3)
        sem_s = (s0, s1, s2, s3)

        pltpu.sync_copy(val_hbm.at[wid], val_v)
        pltpu.sync_copy(dest_hbm.at[wid], dest_v)

        def gath(c, b):
            return pltpu.make_async_copy(
                ff_hbm.at[val_v.at[c]], rows_v.at[b], sem_g[b])

        def scat(c, b):
            return pltpu.make_async_copy(
                rows_v.at[b], out_hbm.at[dest_v.at[c]], sem_s[b])

        for c0 in range(NBUF - 1):
            gath(c0, c0).start()

        def body(t4, carry):
            for b in range(NBUF):
                c = NBUF * t4 + b
                bp = (b + NBUF - 1) % NBUF
                gath(c, b).wait()

                @pl.when(c >= 1)
                def _():
                    scat(c - 1, bp).wait()

                scat(c, b).start()

                @pl.when(c + NBUF - 1 < NCH)
                def _():
                    gath(c + NBUF - 1, bp).start()
            return carry

        lax.fori_loop(0, NCH // NBUF, body, 0)
        scat(NCH - 1, (NCH - 1) % NBUF).wait()

    return gs_kernel(ff, val, dest)


def _tc_mix(ff, nbr, wt, b2):
    def body(x_ref, n_ref, w_ref, b_ref, o_ref):
        y = x_ref[...]
        n0 = n_ref[0]
        n1 = n_ref[1]
        n2 = n_ref[2]
        f1 = n0 + n1 + n2
        f2 = jnp.abs(n2 - n0) + jnp.abs(n0 - n1) + jnp.abs(n1 - n2)
        f3 = jnp.abs(y - n0) + jnp.abs(y - n1) + jnp.abs(y - n2)
        acc = jnp.dot(y, w_ref[0:C, :], preferred_element_type=jnp.float32)
        acc += jnp.dot(f1, w_ref[C:2 * C, :], preferred_element_type=jnp.float32)
        acc += jnp.dot(f2, w_ref[2 * C:3 * C, :], preferred_element_type=jnp.float32)
        acc += jnp.dot(f3, w_ref[3 * C:4 * C, :], preferred_element_type=jnp.float32)
        o_ref[...] = acc + b_ref[...]

    return pl.pallas_call(
        body,
        grid=(F // BF,),
        in_specs=[
            pl.BlockSpec((BF, C), lambda i: (i, 0)),
            pl.BlockSpec((3, BF, C), lambda i: (0, i, 0)),
            pl.BlockSpec((4 * C, C), lambda i: (0, 0)),
            pl.BlockSpec((1, C), lambda i: (0, 0)),
        ],
        out_specs=pl.BlockSpec((BF, C), lambda i: (i, 0)),
        out_shape=jax.ShapeDtypeStruct((F, C), jnp.float32),
    )(ff, nbr, wt, b2)


def kernel(faces, face_features, W, b):
    val, dest = _edge_val_dest(faces)
    val = val.reshape(NW, NCH, CH)
    dest = dest.reshape(NW, NCH, CH)
    nbr = _sc_gather_scatter(face_features, val, dest).reshape(3, FP, C)
    wt = jnp.transpose(W[:, :, 0, :], (2, 1, 0)).reshape(4 * C, C)
    out = _tc_mix(face_features, nbr, wt, b[None, :])
    return out
